# baseline (speedup 1.0000x reference)
import functools

import jax
import jax.numpy as jnp
from jax import lax
from jax.experimental import pallas as pl
from jax.experimental.pallas import tpu as pltpu
from jax.experimental.pallas import tpu_sc as plsc

NC = 2
NS = 16
L = 16
NBUF = 4
G = 2
HB = 80


def _mesh():
    return plsc.VectorSubcoreMesh(core_axis_name="c", subcore_axis_name="s")


def _sc_params():
    return pltpu.CompilerParams(needs_layout_passes=False,
                                use_tc_tiling_on_sc=False)


@functools.cache
def _make_deg_kernel(n, e, eb):
    ept = e // (NC * NS)
    nb = ept // eb
    tr = 632
    assert NS * tr >= n and tr % 8 == 0 and (n - tr) % 8 == 0

    def body(col_hbm, out_hbm, colbuf, ones, zbuf, acc_sh):
        c = lax.axis_index("c")
        s = lax.axis_index("s")
        base = jnp.minimum(s * tr, n - tr)
        zero16 = jnp.zeros((L,), jnp.float32)
        one16 = jnp.ones((L,), jnp.float32)

        def fill(i, _):
            zbuf[i, :] = zero16
            return 0

        lax.fori_loop(0, tr, fill, 0, unroll=4)

        def fill1(i, _):
            ones[i, :] = one16
            return 0

        lax.fori_loop(0, eb, fill1, 0, unroll=4)
        pltpu.sync_copy(zbuf, acc_sh.at[pl.ds(base, tr)])
        plsc.subcore_barrier()

        pltpu.sync_copy(col_hbm.at[c, s], colbuf)

        def batch(i, _):
            pltpu.sync_copy(ones, acc_sh.at[colbuf.at[i]], add=True)
            return 0

        lax.fori_loop(0, nb, batch, 0)
        plsc.subcore_barrier()
        pltpu.sync_copy(acc_sh.at[pl.ds(base, tr)],
                        out_hbm.at[c, pl.ds(base, tr)])

    return pl.kernel(
        body,
        out_type=jax.ShapeDtypeStruct((NC, n, L), jnp.float32),
        mesh=_mesh(),
        compiler_params=_sc_params(),
        scratch_types=[
            pltpu.VMEM((nb, eb), jnp.int32),
            pltpu.VMEM((eb, L), jnp.float32),
            pltpu.VMEM((tr, L), jnp.float32),
            pltpu.VMEM_SHARED((n, L), jnp.float32),
        ],
    )


@functools.cache
def _make_dinv_kernel(n, in_c):
    def body(deg_ref, x_ref, dinv_ref, u0_ref):
        d = deg_ref[0] + deg_ref[1]
        r = lax.rsqrt(jnp.maximum(d, 1.0))
        dinv = jnp.where(d > 0, r, 0.0)[:, :1]
        dinv_ref[...] = dinv
        u0_ref[pl.ds(0, n), :] = x_ref[...] * dinv
        u0_ref[pl.ds(n, 8), :] = jnp.zeros((8, in_c), jnp.float32)

    return pl.pallas_call(
        body,
        out_shape=(
            jax.ShapeDtypeStruct((n, 1), jnp.float32),
            jax.ShapeDtypeStruct((n + 8, in_c), jnp.float32),
        ),
    )


@functools.cache
def _make_scatter_kernel(n, ep, eb, ch):
    ept = ep // (NC * NS)
    nb = ept // eb
    nh = nb // HB
    tr = 632
    assert nb % HB == 0 and HB % NBUF == 0 and NS * tr >= n

    def body(idx_hbm, u_hbm, out_hbm, idxb, r0, r1, r2, r3,
             g0, g1, g2, g3, x0, x1, x2, x3, acc_sh):
        rows = [r0, r1, r2, r3]
        gsem = [g0, g1, g2, g3]
        ssem = [x0, x1, x2, x3]
        c = lax.axis_index("c")
        s = lax.axis_index("s")
        zero16 = jnp.zeros((L,), jnp.float32)
        base = jnp.minimum(s * tr, n - tr)

        def zb(i, _):
            for j in range(ch // L):
                r0[i, pl.ds(j * L, L)] = zero16
            return 0

        lax.fori_loop(0, eb, zb, 0, unroll=4)
        for q in range(tr // eb):
            pltpu.sync_copy(r0, acc_sh.at[pl.ds(base + q * eb, eb)])
        rem = tr - (tr // eb) * eb
        if rem:
            pltpu.sync_copy(r0.at[pl.ds(0, rem)],
                            acc_sh.at[pl.ds(base + (tr // eb) * eb, rem)])
        plsc.subcore_barrier()

        def fire_gather(b, p):
            pltpu.async_copy(u_hbm.at[idxb.at[b, 0]], rows[p], gsem[p])

        for h in range(nh):
            pltpu.sync_copy(idx_hbm.at[c, s, pl.ds(h * HB, HB)], idxb)
            for b in range(G):
                fire_gather(b, b)

            def ring(m, _):
                for p0 in range(NBUF):
                    b = m * NBUF + p0
                    pltpu.make_async_copy(
                        u_hbm.at[idxb.at[b, 0]], rows[p0], gsem[p0]).wait()
                    pltpu.async_copy(rows[p0], acc_sh.at[idxb.at[b, 1]],
                                     ssem[p0], add=True)
                    pp = (p0 + G) % NBUF

                    @pl.when(b >= G)
                    def _():
                        pltpu.make_async_copy(
                            rows[pp], acc_sh.at[idxb.at[b - G, 1]],
                            ssem[pp]).wait()

                    nxt = b + G

                    @pl.when(nxt < HB)
                    def _():
                        fire_gather(nxt, pp)

                return 0

            lax.fori_loop(0, HB // NBUF, ring, 0)
            for j in range(G):
                b = HB - G + j
                pltpu.make_async_copy(
                    rows[b % NBUF], acc_sh.at[idxb.at[b, 1]],
                    ssem[b % NBUF]).wait()

        plsc.subcore_barrier()

        pltpu.sync_copy(acc_sh.at[pl.ds(base, tr)],
                        out_hbm.at[c, pl.ds(base, tr)])

    return pl.kernel(
        body,
        out_type=jax.ShapeDtypeStruct((NC, n, ch), jnp.float32),
        mesh=_mesh(),
        compiler_params=_sc_params(),
        scratch_types=(
            [pltpu.VMEM((HB, 2, eb), jnp.int32)]
            + [pltpu.VMEM((eb, ch), jnp.float32) for _ in range(NBUF)]
            + [pltpu.SemaphoreType.DMA for _ in range(2 * NBUF)]
            + [pltpu.VMEM_SHARED((n, ch), jnp.float32)]),
    )


@functools.cache
def _make_combine_kernel(n, ch, first, last):
    def body(*refs):
        it = iter(refs)
        parts_ref = next(it)
        tpp_ref = None if first else next(it)
        dinv_ref = next(it)
        tk_ref = next(it)
        uk_ref = None if last else next(it)
        dinv = dinv_ref[...]
        p = parts_ref[0] + parts_ref[1]
        if first:
            t = -(dinv * p)
        else:
            t = (-2.0) * (dinv * p) - tpp_ref[...]
        tk_ref[...] = t
        if not last:
            uk_ref[pl.ds(0, n), :] = dinv * t
            uk_ref[pl.ds(n, 8), :] = jnp.zeros((8, ch), jnp.float32)

    outs = [jax.ShapeDtypeStruct((n, ch), jnp.float32)]
    if not last:
        outs.append(jax.ShapeDtypeStruct((n + 8, ch), jnp.float32))
    return pl.pallas_call(
        body,
        out_shape=tuple(outs) if not last else outs[0],
    )


@functools.cache
def _make_matmul_kernel(n, ch, out_c, nk, rb):
    ngrid = n // rb

    def body(*refs):
        t_refs = refs[:nk]
        w_ref, b_ref, o_ref = refs[nk:]
        acc = jnp.broadcast_to(b_ref[...], (rb, out_c))
        for k in range(nk):
            acc = acc + jnp.dot(
                t_refs[k][...], w_ref[k],
                preferred_element_type=jnp.float32,
                precision=lax.Precision.HIGHEST)
        o_ref[...] = acc

    t_spec = pl.BlockSpec((rb, ch), lambda i: (i, 0))
    return pl.pallas_call(
        body,
        grid=(ngrid,),
        in_specs=[t_spec] * nk + [
            pl.BlockSpec((nk, ch, out_c), lambda i: (0, 0, 0)),
            pl.BlockSpec((1, out_c), lambda i: (0, 0)),
        ],
        out_specs=pl.BlockSpec((rb, out_c), lambda i: (i, 0)),
        out_shape=jax.ShapeDtypeStruct((n, out_c), jnp.float32),
    )


def kernel(x, edge_index, W0, W1, W2, b0, b1, b2):
    n, in_c = x.shape
    e = edge_index.shape[1]
    row = edge_index[0]
    col = edge_index[1]

    eb_deg = 80
    col_deg = col.reshape(NC, NS, e // (NC * NS) // eb_deg, eb_deg)
    deg16 = _make_deg_kernel(n, e, eb_deg)(col_deg)
    dinv1, u0 = _make_dinv_kernel(n, in_c)(deg16, x)

    eb = 64
    quant = NC * NS * eb * HB
    ep = ((e + quant - 1) // quant) * quant
    row_p = jnp.concatenate([row, jnp.full((ep - e,), n, jnp.int32)])
    col_p = jnp.concatenate([col, jnp.zeros((ep - e,), jnp.int32)])
    nbt = ep // (NC * NS) // eb
    idx_t = jnp.stack([row_p.reshape(NC, NS, nbt, eb),
                       col_p.reshape(NC, NS, nbt, eb)], axis=3)

    kmax = max(W0.shape[0], W1.shape[0], W2.shape[0])
    scat = _make_scatter_kernel(n, ep, eb, in_c)
    ts = [x]
    u_prev = u0
    for k in range(1, kmax):
        first = k == 1
        last = k == kmax - 1
        parts = scat(idx_t, u_prev)
        comb = _make_combine_kernel(n, in_c, first, last)
        args = [parts] if first else [parts, ts[-2]]
        res = comb(*args, dinv1)
        if last:
            ts.append(res)
        else:
            tk, uk = res
            ts.append(tk)
            u_prev = uk

    out_c = W0.shape[2] + W1.shape[2] + W2.shape[2]
    wblocks = []
    for k in range(kmax):
        cols = []
        for W in (W0, W1, W2):
            if k < W.shape[0]:
                cols.append(W[k])
            else:
                cols.append(jnp.zeros((in_c, W.shape[2]), jnp.float32))
        wblocks.append(jnp.concatenate(cols, axis=1))
    wbig = jnp.stack(wblocks)
    bias = jnp.concatenate([b0, b1, b2])[None, :]

    mm = _make_matmul_kernel(n, in_c, out_c, kmax, 1000)
    return mm(*ts, wbig, bias)


# speedup vs baseline: 1.0457x; 1.0457x over previous
import functools

import jax
import jax.numpy as jnp
from jax import lax
from jax.experimental import pallas as pl
from jax.experimental.pallas import tpu as pltpu
from jax.experimental.pallas import tpu_sc as plsc

NC = 2
NS = 16
L = 16
NBUF = 4
G = 2


def _mesh():
    return plsc.VectorSubcoreMesh(core_axis_name="c", subcore_axis_name="s")


def _sc_params():
    return pltpu.CompilerParams(needs_layout_passes=False,
                                use_tc_tiling_on_sc=False)


@functools.cache
def _make_deg_kernel(n, e, eb):
    ept = e // (NC * NS)
    nb = ept // eb
    tr = 632
    assert NS * tr >= n and tr % 8 == 0 and (n - tr) % 8 == 0

    def body(col_hbm, out_hbm, colbuf, ones, zbuf, *rest):
        dsem = list(rest[:8])
        acc_sh = rest[8]
        c = lax.axis_index("c")
        s = lax.axis_index("s")
        base = jnp.minimum(s * tr, n - tr)
        zero16 = jnp.zeros((L,), jnp.float32)
        one16 = jnp.ones((L,), jnp.float32)

        def fill(i, _):
            zbuf[i, :] = zero16
            return 0

        lax.fori_loop(0, tr, fill, 0, unroll=4)

        def fill1(i, _):
            ones[i, :] = one16
            return 0

        lax.fori_loop(0, eb, fill1, 0, unroll=4)
        pltpu.sync_copy(zbuf, acc_sh.at[pl.ds(base, tr)])
        plsc.subcore_barrier()

        pltpu.sync_copy(col_hbm.at[c, s], colbuf)

        def fire(i, p):
            pltpu.async_copy(ones, acc_sh.at[colbuf.at[i]], dsem[p],
                             add=True)

        def drain(i, p):
            pltpu.make_async_copy(ones, acc_sh.at[colbuf.at[i]],
                                  dsem[p]).wait()

        ND = 8

        def batch(m, _):
            for p in range(ND):
                i = m * ND + p

                @pl.when(i >= ND)
                def _():
                    drain(i - ND, p)

                fire(i, p)
            return 0

        lax.fori_loop(0, nb // ND, batch, 0)
        for j in range((nb // ND) * ND, nb):
            drain(j - ND, j % ND)
            fire(j, j % ND)
        for b in range(nb - ND, nb):
            drain(b, b % ND)
        plsc.subcore_barrier()
        pltpu.sync_copy(acc_sh.at[pl.ds(base, tr)],
                        out_hbm.at[c, pl.ds(base, tr)])

    return pl.kernel(
        body,
        out_type=jax.ShapeDtypeStruct((NC, n, L), jnp.float32),
        mesh=_mesh(),
        compiler_params=_sc_params(),
        scratch_types=[
            pltpu.VMEM((nb, eb), jnp.int32),
            pltpu.VMEM((eb, L), jnp.float32),
            pltpu.VMEM((tr, L), jnp.float32),
        ] + [pltpu.SemaphoreType.DMA for _ in range(8)] + [
            pltpu.VMEM_SHARED((n, L), jnp.float32),
        ],
    )


@functools.cache
def _make_dinv_kernel(n, in_c):
    def body(deg_ref, x_ref, dinv_ref, u0_ref):
        d = deg_ref[0] + deg_ref[1]
        r = lax.rsqrt(jnp.maximum(d, 1.0))
        dinv = jnp.where(d > 0, r, 0.0)
        dinv_ref[...] = dinv
        u0_ref[...] = x_ref[...] * dinv[:, :1]

    return pl.pallas_call(
        body,
        out_shape=(
            jax.ShapeDtypeStruct((n, L), jnp.float32),
            jax.ShapeDtypeStruct((n, in_c), jnp.float32),
        ),
    )


@functools.cache
def _make_prop_kernel(n, ep, eb, ch, first, last):
    ept = ep // NS
    nb = ept // eb
    ng = eb // L
    tr, cb = 640, 80
    ncb = tr // cb
    assert NS * tr >= n and nb % (2 * NBUF) == 0

    def body(*refs):
        it = iter(refs)
        idx_hbm = next(it)
        u_hbm = next(it)
        tpp_hbm = None if first else next(it)
        dinv_hbm = next(it)
        tk_hbm = next(it)
        uk_hbm = None if last else next(it)
        idx_all = next(it)
        rows = [next(it) for _ in range(NBUF)]
        gsem = [next(it) for _ in range(NBUF)]
        ssem = [next(it) for _ in range(NBUF)]
        abuf, bbuf, dinvbuf, acc_sh = (
            next(it), next(it), next(it), next(it))

        c = lax.axis_index("c")
        s = lax.axis_index("s")
        zero16 = jnp.zeros((L,), jnp.float32)
        base = jnp.minimum(s * tr, n - tr)

        def zb(i, _):
            for j in range(ch // L):
                abuf[i, pl.ds(j * L, L)] = zero16
            return 0

        lax.fori_loop(0, cb, zb, 0, unroll=4)
        for k2 in range(ncb):
            pltpu.sync_copy(abuf, acc_sh.at[pl.ds(base + k2 * cb, cb)])
        plsc.subcore_barrier()

        pltpu.sync_copy(idx_hbm.at[c, s], idx_all)

        def fire_gather(b, p):
            pltpu.async_copy(u_hbm.at[idx_all.at[b, 0]], rows[p], gsem[p])

        def wait_gather(b, p):
            pltpu.make_async_copy(
                u_hbm.at[idx_all.at[b, 0]], rows[p], gsem[p]).wait()

        for b in range(G):
            fire_gather(b, b)

        def ring(m, _):
            for p0 in range(NBUF):
                i = m * NBUF + p0
                wait_gather(i, p0)
                pltpu.async_copy(rows[p0], acc_sh.at[idx_all.at[i, 1]],
                                 ssem[p0], add=True)
                pp = (p0 + G) % NBUF

                @pl.when(i >= G)
                def _():
                    pltpu.make_async_copy(
                        rows[pp], acc_sh.at[idx_all.at[i - G, 1]],
                        ssem[pp]).wait()

                nxt = i + G

                @pl.when(nxt < nb)
                def _():
                    fire_gather(nxt, pp)

            return 0

        lax.fori_loop(0, nb // NBUF, ring, 0)
        for j in range(G):
            b = nb - G + j
            pltpu.make_async_copy(
                rows[b % NBUF], acc_sh.at[idx_all.at[b, 1]],
                ssem[b % NBUF]).wait()
        plsc.subcore_barrier()

        pltpu.sync_copy(dinv_hbm.at[pl.ds(base, tr)], dinvbuf)
        iota = lax.iota(jnp.int32, L)
        for k2 in range(ncb):
            off = base + k2 * cb
            pltpu.sync_copy(acc_sh.at[pl.ds(off, cb)], abuf)
            if not first:
                pltpu.sync_copy(tpp_hbm.at[pl.ds(c * n + off, cb)], bbuf)

            def cmb(g, _):
                ridx = iota + g * L
                dv = dinvbuf[pl.ds(k2 * cb + g * L, L)]
                for j in range(ch):
                    cidx = jnp.full((L,), j, jnp.int32)
                    a = plsc.load_gather(abuf, [ridx, cidx])
                    if first:
                        t = -(dv * a)
                    else:
                        b = plsc.load_gather(bbuf, [ridx, cidx])
                        t = (-2.0) * (dv * a) - b
                    plsc.store_scatter(abuf, [ridx, cidx], t)
                    if not last:
                        plsc.store_scatter(bbuf, [ridx, cidx], dv * t)
                return 0

            lax.fori_loop(0, cb // L, cmb, 0)
            pltpu.sync_copy(abuf, tk_hbm.at[pl.ds(c * n + off, cb)])
            if not last:
                pltpu.sync_copy(bbuf, uk_hbm.at[pl.ds(c * n + off, cb)])

    n_out = 1 if last else 2
    out_type = [jax.ShapeDtypeStruct((NC * n, ch), jnp.float32)] * n_out
    return pl.kernel(
        body,
        out_type=out_type if n_out > 1 else out_type[0],
        mesh=_mesh(),
        compiler_params=_sc_params(),
        scratch_types=(
            [pltpu.VMEM((nb, 2, eb), jnp.int32)]
            + [pltpu.VMEM((eb, ch), jnp.float32) for _ in range(NBUF)]
            + [pltpu.SemaphoreType.DMA for _ in range(2 * NBUF)] + [
                pltpu.VMEM((80, ch), jnp.float32),
                pltpu.VMEM((80, ch), jnp.float32),
                pltpu.VMEM((640,), jnp.float32),
                pltpu.VMEM_SHARED((n + 8, ch), jnp.float32),
            ]),
    )


@functools.cache
def _make_matmul_kernel(n, ch, out_c, nk, rb):
    ngrid = n // rb

    def body(*refs):
        t_refs = refs[:nk]
        w_ref, b_ref, o_ref = refs[nk:]
        acc = jnp.broadcast_to(b_ref[...], (rb, out_c))
        for k in range(nk):
            for c in range(NC):
                acc = acc + jnp.dot(
                    t_refs[k][c], w_ref[k * NC + c],
                    preferred_element_type=jnp.float32,
                    precision=lax.Precision.HIGHEST)
        o_ref[...] = acc

    t_spec = pl.BlockSpec((NC, rb, ch), lambda i: (0, i, 0))
    return pl.pallas_call(
        body,
        grid=(ngrid,),
        in_specs=[t_spec] * nk + [
            pl.BlockSpec((nk * NC, ch, out_c), lambda i: (0, 0, 0)),
            pl.BlockSpec((1, out_c), lambda i: (0, 0)),
        ],
        out_specs=pl.BlockSpec((rb, out_c), lambda i: (i, 0)),
        out_shape=jax.ShapeDtypeStruct((n, out_c), jnp.float32),
    )


def kernel(x, edge_index, W0, W1, W2, b0, b1, b2):
    n, in_c = x.shape
    e = edge_index.shape[1]
    ch = in_c // NC
    row = edge_index[0]
    col = edge_index[1]

    eb_deg = 80
    col_deg = col.reshape(NC, NS, e // (NC * NS) // eb_deg, eb_deg)
    deg16 = _make_deg_kernel(n, e, eb_deg)(col_deg)
    dinv16, u0 = _make_dinv_kernel(n, in_c)(deg16, x)
    dinv = dinv16[:, 0]

    eb = 128
    quant = NS * eb * 2 * NBUF
    ep = ((e + quant - 1) // quant) * quant
    row_p = jnp.concatenate([row, jnp.zeros((ep - e,), jnp.int32)])
    col_p = jnp.concatenate([col, jnp.full((ep - e,), n, jnp.int32)])
    nbt = ep // NS // eb
    idx_t = jnp.stack(
        [jnp.stack([(row_p + c * n).reshape(NS, nbt, eb),
                    col_p.reshape(NS, nbt, eb)], axis=2)
         for c in range(NC)])

    t0 = x.reshape(n, NC, ch).transpose(1, 0, 2).reshape(NC * n, ch)
    u0 = u0.reshape(n, NC, ch).transpose(1, 0, 2).reshape(NC * n, ch)

    kmax = max(W0.shape[0], W1.shape[0], W2.shape[0])
    ts = [t0]
    us = [u0]
    for k in range(1, kmax):
        first = k == 1
        last = k == kmax - 1
        prop = _make_prop_kernel(n, ep, eb, ch, first, last)
        args = [idx_t, us[-1]]
        if not first:
            args.append(ts[-2])
        args.append(dinv)
        res = prop(*args)
        if last:
            ts.append(res)
        else:
            tk, uk = res
            ts.append(tk)
            us.append(uk)

    out_c = W0.shape[2] + W1.shape[2] + W2.shape[2]
    wblocks = []
    for k in range(kmax):
        for c in range(NC):
            cols = []
            for W in (W0, W1, W2):
                if k < W.shape[0]:
                    cols.append(W[k, c * ch:(c + 1) * ch, :])
                else:
                    cols.append(jnp.zeros((ch, W.shape[2]), jnp.float32))
            wblocks.append(jnp.concatenate(cols, axis=1))
    wbig = jnp.stack(wblocks)
    bias = jnp.concatenate([b0, b1, b2])[None, :]

    t_in = [t.reshape(NC, n, ch) for t in ts]
    mm = _make_matmul_kernel(n, ch, out_c, kmax, 1000)
    return mm(*t_in, wbig, bias)
